# baseline (speedup 1.0000x reference)
import functools

import jax
import jax.numpy as jnp
import numpy as np
from jax import lax
from jax.experimental import pallas as pl
from jax.experimental.pallas import tpu as pltpu
from jax.experimental.pallas import tpu_sc as plsc

MAX_Z = 36
N_ORB = 13
EMBED_DIM = 64
SUB_CAPS = [2, 2, 3, 3, 2, 3, 3, 2, 4, 3, 3, 3, 3]

NC, NS = 2, 16
NW = NC * NS
ROW = N_ORB * EMBED_DIM
DPW = EMBED_DIM // NW
RPW = N_ORB * DPW
ZSTR = 48


def _elec_idx_const() -> np.ndarray:
    t = np.zeros((MAX_Z + 1, N_ORB), dtype=np.int32)
    for zz in range(1, MAX_Z + 1):
        rem = zz
        for col, cap in enumerate(SUB_CAPS):
            e = min(rem, cap)
            t[zz, col] = e
            rem -= e
            if rem == 0:
                break
    idx2 = np.zeros(N_ORB * ZSTR + 16, dtype=np.int32)
    for o in range(N_ORB):
        idx2[o * ZSTR: o * ZSTR + MAX_Z + 1] = o * 5 + t[:, o]
    return idx2


_MESH = plsc.VectorSubcoreMesh(core_axis_name="c", subcore_axis_name="s")
_SC_PARAMS = pltpu.CompilerParams(
    use_tc_tiling_on_sc=False, needs_layout_passes=False)


def _combine_body(nlr, chrows, z_hbm, zet_hbm, ef_hbm, idx_hbm, out_hbm,
                  emb_v, idx_v, ftw_v, z_v, ze_v, obuf, sem_z, sem_e, sem_o):
    wid = lax.axis_index("s") * NC + lax.axis_index("c")
    nchunks = nlr // chrows
    chn = chrows * 128

    pltpu.sync_copy(ef_hbm, emb_v)
    pltpu.sync_copy(idx_hbm, idx_v)

    for r in range(RPW):
        o, di = r // DPW, r % DPW
        dvec = jnp.broadcast_to(DPW * wid + di, (16,)).astype(jnp.int32)
        for ch in range(3):
            rv = idx_v[pl.ds(o * ZSTR + ch * 16, 16)]
            ftw_v[pl.ds(r * ZSTR + ch * 16, 16)] = plsc.load_gather(
                emb_v, [rv, dvec])

    cvec = [jnp.full((16,), r * ZSTR, jnp.int32) for r in range(RPW)]

    def issue_in(i):
        @pl.when(i < nchunks)
        def _():
            p = lax.rem(i, 2)
            pltpu.async_copy(
                z_hbm.at[pl.ds(i * chn, chn)], z_v.at[p], sem_z)
            for di in range(DPW):
                d = DPW * wid + di
                pltpu.async_copy(
                    zet_hbm.at[pl.ds(d * nlr + i * chrows, chrows)],
                    ze_v.at[p].at[di], sem_e)

    def compute(p):
        @plsc.parallel_loop(0, chrows, unroll=2)
        def lrow(gr):
            for gc in range(8):
                zvec = z_v[p, pl.ds(gr * 128 + gc * 16, 16)]
                m = []
                for di in range(DPW):
                    m.append(ze_v[p, di, gr, pl.ds(gc * 16, 16)] + 1.0)
                for r in range(RPW):
                    obuf[r, gr, pl.ds(gc * 16, 16)] = (
                        plsc.load_gather(ftw_v, [cvec[r] + zvec]) * m[r % DPW]
                    )

    def out_row(r):
        return ((r // DPW) * EMBED_DIM + DPW * wid + r % DPW) * nlr

    issue_in(0)

    def chunk(i, carry):
        p = lax.rem(i, 2)
        issue_in(i + 1)
        pltpu.make_async_copy(
            z_hbm.at[pl.ds(i * chn, chn)], z_v.at[p], sem_z).wait()
        for di in range(DPW):
            d = DPW * wid + di
            pltpu.make_async_copy(
                zet_hbm.at[pl.ds(d * nlr + i * chrows, chrows)],
                ze_v.at[p].at[di], sem_e).wait()

        @pl.when(i >= 1)
        def _():
            for r in range(RPW):
                pltpu.make_async_copy(
                    obuf.at[r],
                    out_hbm.at[pl.ds(out_row(r) + (i - 1) * chrows, chrows)],
                    sem_o).wait()

        compute(p)
        for r in range(RPW):
            pltpu.async_copy(
                obuf.at[r],
                out_hbm.at[pl.ds(out_row(r) + i * chrows, chrows)],
                sem_o)
        return carry

    lax.fori_loop(0, nchunks, chunk, 0)

    for r in range(RPW):
        pltpu.make_async_copy(
            obuf.at[r],
            out_hbm.at[pl.ds(out_row(r) + (nchunks - 1) * chrows, chrows)],
            sem_o).wait()


def _make_combine(nlr, chrows):
    return pl.kernel(
        functools.partial(_combine_body, nlr, chrows),
        out_type=jax.ShapeDtypeStruct((ROW * nlr, 128), jnp.float32),
        mesh=_MESH,
        compiler_params=_SC_PARAMS,
        scratch_types=[
            pltpu.VMEM((N_ORB * 5 + 7, 128), jnp.float32),
            pltpu.VMEM((N_ORB * ZSTR + 16,), jnp.int32),
            pltpu.VMEM((RPW * ZSTR,), jnp.float32),
            pltpu.VMEM((2, chrows * 128), jnp.int32),
            pltpu.VMEM((2, DPW, chrows, 128), jnp.float32),
            pltpu.VMEM((RPW, chrows, 128), jnp.float32),
            pltpu.SemaphoreType.DMA,
            pltpu.SemaphoreType.DMA,
            pltpu.SemaphoreType.DMA,
        ],
    )


def kernel(z, z_embed, embeds):
    n_node = z.shape[0]
    npad = -(-n_node // 128) * 128
    nlr = npad // 128
    chrows = 17 if nlr % 17 == 0 else 1
    assert nlr % chrows == 0
    z32 = jnp.pad(z.astype(jnp.int32), (0, npad - n_node))
    zet = jnp.pad(z_embed.astype(jnp.float32).T, ((0, 0), (0, npad - n_node)))
    zet2 = zet.reshape(EMBED_DIM * nlr, 128)
    ef2 = jnp.pad(
        embeds.astype(jnp.float32).reshape(N_ORB * 5, EMBED_DIM),
        ((0, 7), (0, EMBED_DIM)))
    idx2 = jnp.asarray(_elec_idx_const())
    out = _make_combine(nlr, chrows)(z32, zet2, ef2, idx2)
    out3 = jnp.transpose(out.reshape(N_ORB, EMBED_DIM, npad), (2, 0, 1))
    return out3[:n_node]


# speedup vs baseline: 2.3832x; 2.3832x over previous
import functools

import jax
import jax.numpy as jnp
import numpy as np
from jax import lax
from jax.experimental import pallas as pl
from jax.experimental.pallas import tpu as pltpu
from jax.experimental.pallas import tpu_sc as plsc

MAX_Z = 36
N_ORB = 13
EMBED_DIM = 64
SUB_CAPS = [2, 2, 3, 3, 2, 3, 3, 2, 4, 3, 3, 3, 3]

NC, NS = 2, 16
NW = NC * NS
ROW = N_ORB * EMBED_DIM
DPW = EMBED_DIM // NW
RPW = N_ORB * DPW
ZSTR = 48


def _elec_idx_const() -> np.ndarray:
    t = np.zeros((MAX_Z + 1, N_ORB), dtype=np.int32)
    for zz in range(1, MAX_Z + 1):
        rem = zz
        for col, cap in enumerate(SUB_CAPS):
            e = min(rem, cap)
            t[zz, col] = e
            rem -= e
            if rem == 0:
                break
    idx2 = np.zeros(N_ORB * ZSTR + 16, dtype=np.int32)
    for o in range(N_ORB):
        idx2[o * ZSTR: o * ZSTR + MAX_Z + 1] = o * 5 + t[:, o]
    return idx2


_MESH = plsc.VectorSubcoreMesh(core_axis_name="c", subcore_axis_name="s")
_SC_PARAMS = pltpu.CompilerParams(
    use_tc_tiling_on_sc=False, needs_layout_passes=False)


def _combine_body(nlr, chrows, z_hbm, zet_hbm, ef_hbm, idx_hbm, out_hbm,
                  emb_v, idx_v, ftw_v, z_v, ze_v, obuf, sem_z, sem_e, sem_o):
    wid = lax.axis_index("s") * NC + lax.axis_index("c")
    nchunks = nlr // chrows
    chn = chrows * 128

    pltpu.sync_copy(ef_hbm, emb_v)
    pltpu.sync_copy(idx_hbm, idx_v)

    for r in range(RPW):
        o, di = r // DPW, r % DPW
        dvec = jnp.broadcast_to(DPW * wid + di, (16,)).astype(jnp.int32)
        for ch in range(3):
            rv = idx_v[pl.ds(o * ZSTR + ch * 16, 16)]
            ftw_v[pl.ds(r * ZSTR + ch * 16, 16)] = plsc.load_gather(
                emb_v, [rv, dvec])

    cvec = [jnp.full((16,), r * ZSTR, jnp.int32) for r in range(RPW)]

    def issue_in(i):
        @pl.when(i < nchunks)
        def _():
            p = lax.rem(i, 2)
            pltpu.async_copy(
                z_hbm.at[pl.ds(i * chn, chn)], z_v.at[p], sem_z)
            for di in range(DPW):
                d = DPW * wid + di
                pltpu.async_copy(
                    zet_hbm.at[pl.ds(d * nlr + i * chrows, chrows)],
                    ze_v.at[p].at[di], sem_e)

    def compute(p):
        @plsc.parallel_loop(0, chrows)
        def lrow(gr):
            for gc in range(8):
                zvec = z_v[p, pl.ds(gr * 128 + gc * 16, 16)]
                m = []
                for di in range(DPW):
                    m.append(ze_v[p, di, gr, pl.ds(gc * 16, 16)] + 1.0)
                for r in range(RPW):
                    obuf[r, gr, pl.ds(gc * 16, 16)] = (
                        plsc.load_gather(ftw_v, [cvec[r] + zvec]) * m[r % DPW]
                    )

    def out_row(r):
        return ((r // DPW) * EMBED_DIM + DPW * wid + r % DPW) * nlr

    issue_in(0)

    def chunk(i, carry):
        p = lax.rem(i, 2)
        issue_in(i + 1)
        pltpu.make_async_copy(
            z_hbm.at[pl.ds(i * chn, chn)], z_v.at[p], sem_z).wait()
        for di in range(DPW):
            d = DPW * wid + di
            pltpu.make_async_copy(
                zet_hbm.at[pl.ds(d * nlr + i * chrows, chrows)],
                ze_v.at[p].at[di], sem_e).wait()

        @pl.when(i >= 1)
        def _():
            for r in range(RPW):
                pltpu.make_async_copy(
                    obuf.at[r],
                    out_hbm.at[pl.ds(out_row(r) + (i - 1) * chrows, chrows)],
                    sem_o).wait()

        compute(p)
        for r in range(RPW):
            pltpu.async_copy(
                obuf.at[r],
                out_hbm.at[pl.ds(out_row(r) + i * chrows, chrows)],
                sem_o)
        return carry

    lax.fori_loop(0, nchunks, chunk, 0)

    for r in range(RPW):
        pltpu.make_async_copy(
            obuf.at[r],
            out_hbm.at[pl.ds(out_row(r) + (nchunks - 1) * chrows, chrows)],
            sem_o).wait()


def _make_combine(nlr, chrows):
    return pl.kernel(
        functools.partial(_combine_body, nlr, chrows),
        out_type=jax.ShapeDtypeStruct((ROW * nlr, 128), jnp.float32),
        mesh=_MESH,
        compiler_params=_SC_PARAMS,
        scratch_types=[
            pltpu.VMEM((N_ORB * 5 + 7, 128), jnp.float32),
            pltpu.VMEM((N_ORB * ZSTR + 16,), jnp.int32),
            pltpu.VMEM((RPW * ZSTR,), jnp.float32),
            pltpu.VMEM((2, chrows * 128), jnp.int32),
            pltpu.VMEM((2, DPW, chrows, 128), jnp.float32),
            pltpu.VMEM((RPW, chrows, 128), jnp.float32),
            pltpu.SemaphoreType.DMA,
            pltpu.SemaphoreType.DMA,
            pltpu.SemaphoreType.DMA,
        ],
    )


def kernel(z, z_embed, embeds):
    n_node = z.shape[0]
    npad = -(-n_node // 128) * 128
    nlr = npad // 128
    chrows = 23 if nlr % 23 == 0 else 1
    assert nlr % chrows == 0
    z32 = jnp.pad(z.astype(jnp.int32), (0, npad - n_node))
    zet = jnp.pad(z_embed.astype(jnp.float32).T, ((0, 0), (0, npad - n_node)))
    zet2 = zet.reshape(EMBED_DIM * nlr, 128)
    ef2 = jnp.pad(
        embeds.astype(jnp.float32).reshape(N_ORB * 5, EMBED_DIM),
        ((0, 7), (0, EMBED_DIM)))
    idx2 = jnp.asarray(_elec_idx_const())
    out = _make_combine(nlr, chrows)(z32, zet2, ef2, idx2)
    out3 = jnp.transpose(out.reshape(N_ORB, EMBED_DIM, npad), (2, 0, 1))
    return out3[:n_node]
